# baseline (speedup 1.0000x reference)
import functools

import jax
import jax.numpy as jnp
from jax import lax
from jax.experimental import pallas as pl
from jax.experimental.pallas import tpu as pltpu
from jax.experimental.pallas import tpu_sc as plsc

N = 10000
E = 320000
IN_CH = 128
HID = 16
HEADS = 8
OUT_CH = 64

NB = 400


def _head_expand_mat(att):
    hh, cc = att.shape
    hc = hh * cc
    j = lax.broadcasted_iota(jnp.int32, (hc, hh), 0)
    k = lax.broadcasted_iota(jnp.int32, (hc, hh), 1)
    return jnp.where(j // cc == k, att.reshape(hc)[:, None], 0.0)


def _tc1_body(x_ref, w_ref, msrc_ref, mdst_ref, h_ref, acat_ref):
    xb = x_ref[...]
    h1 = xb @ w_ref[...]
    a_src = h1 @ msrc_ref[...]
    a_dst = h1 @ mdst_ref[...]
    h_ref[...] = jnp.concatenate([h1, a_src], axis=1)
    acat_ref[...] = jnp.concatenate([a_src, a_dst], axis=1)


def _tc1(x, W1, att_src1, att_dst1):
    msrc = _head_expand_mat(att_src1)
    mdst = _head_expand_mat(att_dst1)
    grid = (N // NB,)
    return pl.pallas_call(
        _tc1_body,
        grid=grid,
        in_specs=[
            pl.BlockSpec((NB, IN_CH), lambda i: (i, 0)),
            pl.BlockSpec((IN_CH, HEADS * HID), lambda i: (0, 0)),
            pl.BlockSpec((HEADS * HID, HEADS), lambda i: (0, 0)),
            pl.BlockSpec((HEADS * HID, HEADS), lambda i: (0, 0)),
        ],
        out_specs=[
            pl.BlockSpec((NB, HEADS * HID + 8), lambda i: (i, 0)),
            pl.BlockSpec((NB, 2 * HEADS), lambda i: (i, 0)),
        ],
        out_shape=[
            jax.ShapeDtypeStruct((N, HEADS * HID + 8), jnp.float32),
            jax.ShapeDtypeStruct((N, 2 * HEADS), jnp.float32),
        ],
    )(x, W1, msrc, mdst)


def _tc2_body(a0_ref, a1_ref, b1_ref, w2_ref, as2_ref, ad2_ref,
              h2_ref, acat2_ref):
    acc = a0_ref[...] + a1_ref[...]
    msg = acc[:, : HEADS * HID]
    wsum = acc[:, HEADS * HID : HEADS * HID + HEADS]
    j = lax.broadcasted_iota(jnp.int32, (HEADS, HEADS * HID), 1)
    hh = lax.broadcasted_iota(jnp.int32, (HEADS, HEADS * HID), 0)
    rep = jnp.where(j // HID == hh, 1.0, 0.0)
    denom = wsum @ rep
    x1 = msg / (denom + 1e-16) + b1_ref[...][None, :]
    xe = jnp.where(x1 > 0, x1, jnp.exp(x1) - 1.0)
    h2 = xe @ w2_ref[...]
    a2s = h2 @ as2_ref[...]
    a2d = h2 @ ad2_ref[...]
    pad = jnp.zeros((a2s.shape[0], 14), jnp.float32)
    h2_ref[...] = jnp.concatenate([h2, a2s, a2d, pad], axis=1)
    acat2_ref[...] = jnp.concatenate([a2s, a2d, pad], axis=1)


def _tc2(acc0, acc1, b1, W2, att_src2, att_dst2):
    grid = (N // NB,)
    d1 = HEADS * HID + HEADS
    return pl.pallas_call(
        _tc2_body,
        grid=grid,
        in_specs=[
            pl.BlockSpec((NB, d1), lambda i: (i, 0)),
            pl.BlockSpec((NB, d1), lambda i: (i, 0)),
            pl.BlockSpec((HEADS * HID,), lambda i: (0,)),
            pl.BlockSpec((HEADS * HID, OUT_CH), lambda i: (0, 0)),
            pl.BlockSpec((OUT_CH, 1), lambda i: (0, 0)),
            pl.BlockSpec((OUT_CH, 1), lambda i: (0, 0)),
        ],
        out_specs=[
            pl.BlockSpec((NB, OUT_CH + 16), lambda i: (i, 0)),
            pl.BlockSpec((NB, 16), lambda i: (i, 0)),
        ],
        out_shape=[
            jax.ShapeDtypeStruct((N, OUT_CH + 16), jnp.float32),
            jax.ShapeDtypeStruct((N, 16), jnp.float32),
        ],
    )(acc0, acc1, b1, W2,
      att_src2.reshape(OUT_CH, 1), att_dst2.reshape(OUT_CH, 1))


def _tc3_body(a0_ref, a1_ref, b2_ref, out_ref):
    acc = a0_ref[...] + a1_ref[...]
    out = acc[:, :OUT_CH] / (acc[:, OUT_CH : OUT_CH + 1] + 1e-16) + b2_ref[...][None, :]
    m = jnp.max(out, axis=1, keepdims=True)
    lse = jnp.log(jnp.sum(jnp.exp(out - m), axis=1, keepdims=True)) + m
    out_ref[...] = out - lse


def _tc3(acc0, acc1, b2):
    grid = (N // NB,)
    d2 = OUT_CH + 16
    return pl.pallas_call(
        _tc3_body,
        grid=grid,
        in_specs=[
            pl.BlockSpec((NB, d2), lambda i: (i, 0)),
            pl.BlockSpec((NB, d2), lambda i: (i, 0)),
            pl.BlockSpec((OUT_CH,), lambda i: (0,)),
        ],
        out_specs=pl.BlockSpec((NB, OUT_CH), lambda i: (i, 0)),
        out_shape=jax.ShapeDtypeStruct((N, OUT_CH), jnp.float32),
    )(acc0, acc1, b2)


NC = 2
NS = 16
NW = NC * NS
K1 = 40
NBLK1 = E // (NW * K1)
K2 = 80
NBLK2 = E // (NW * K2)
RPT = 624
RTAIL = N - NS * RPT

_SC_MESH = dict(core_axis_name="c", subcore_axis_name="s")
_SC_PARAMS = pltpu.CompilerParams(use_tc_tiling_on_sc=False,
                                  needs_layout_passes=False)


def _edge_pass_sc_l1(h_mat, acat, src_r, dst_r):
    mesh = plsc.VectorSubcoreMesh(**_SC_MESH)
    NR = 4

    @functools.partial(
        pl.kernel,
        out_type=jax.ShapeDtypeStruct((NC, N, 136), jnp.float32),
        mesh=mesh,
        compiler_params=_SC_PARAMS,
        scratch_types=[
            pltpu.VMEM((NBLK1, K1), jnp.int32),
            pltpu.VMEM((NBLK1, K1), jnp.int32),
        ] + [pltpu.VMEM((K1, 136), jnp.float32)] * NR
          + [pltpu.VMEM((K1, 16), jnp.float32)] * NR
          + [pltpu.VMEM_SHARED((N, 136), jnp.float32)]
          + [pltpu.SemaphoreType.DMA] * (3 * NR),
    )
    def k(h_hbm, acat_hbm, srcr_hbm, dstr_hbm, out_hbm, srcv, dstv, *rest):
        rows = rest[0:NR]
        bvs = rest[NR:2 * NR]
        acc = rest[2 * NR]
        grs = rest[2 * NR + 1: 2 * NR + 1 + NR]
        gbs = rest[2 * NR + 1 + NR: 2 * NR + 1 + 2 * NR]
        sms = rest[2 * NR + 1 + 2 * NR: 2 * NR + 1 + 3 * NR]
        c = lax.axis_index("c")
        s = lax.axis_index("s")
        wid = s * NC + c
        lane = lax.iota(jnp.int32, 16)
        lane8 = lane // 8
        head8 = lane - 8 * lane8

        zv = jnp.zeros((16,), jnp.float32)

        def zbuf(i, _):
            for j in range(8):
                rows[0][i, pl.ds(j * 16, 16)] = zv
            rows[0][i, pl.ds(120, 16)] = zv
            return 0

        lax.fori_loop(0, K1, zbuf, 0)
        for t in range(15):
            pltpu.sync_copy(rows[0], acc.at[pl.ds(s * RPT + t * K1, K1)])
        pltpu.sync_copy(rows[0].at[pl.ds(0, RPT - 15 * K1)],
                        acc.at[pl.ds(s * RPT + 15 * K1, RPT - 15 * K1)])

        @pl.when(s == NS - 1)
        def _():
            pltpu.sync_copy(rows[0].at[pl.ds(0, RTAIL)],
                            acc.at[pl.ds(NS * RPT, RTAIL)])

        plsc.subcore_barrier()

        pltpu.sync_copy(srcr_hbm.at[wid], srcv)
        pltpu.sync_copy(dstr_hbm.at[wid], dstv)

        def start_gather(b, r):
            pltpu.async_copy(h_hbm.at[srcv.at[b]], rows[r], grs[r])
            pltpu.async_copy(acat_hbm.at[dstv.at[b]], bvs[r], gbs[r])

        def wait_gather(r):
            pltpu.make_async_copy(h_hbm.at[pl.ds(0, K1)], rows[r], grs[r]).wait()
            pltpu.make_async_copy(acat_hbm.at[pl.ds(0, K1)], bvs[r], gbs[r]).wait()

        def start_scatter(b, r):
            pltpu.async_copy(rows[r], acc.at[dstv.at[b]], sms[r], add=True)

        def wait_scatter(r):
            pltpu.make_async_copy(rows[r], acc.at[pl.ds(0, K1)], sms[r]).wait()

        def compute(r):
            rw = rows[r]
            bv = bvs[r]

            def blk(i2, _):
                eids = lane8 + 2 * i2
                asr = plsc.load_gather(rw, [eids, head8 + 128])
                ads = plsc.load_gather(bv, [eids, head8 + 8])
                e = asr + ads
                e = jnp.where(e >= 0, e, e * 0.2)
                w = jnp.exp(e)
                plsc.store_scatter(rw, [eids, head8 + 128], w)
                for d in range(2):
                    i = 2 * i2 + d
                    for j in range(8):
                        w_s = w[d * 8 + j]
                        rw[i, pl.ds(j * 16, 16)] = rw[i, pl.ds(j * 16, 16)] * w_s
                return 0

            lax.fori_loop(0, K1 // 2, blk, 0)

        for r in range(NR):
            start_gather(r, r)

        NQ = NBLK1 // NR
        TAIL = NBLK1 - NR * NQ

        def quad(q, _):
            for r in range(NR):
                b = NR * q + r
                wait_gather(r)

                @pl.when(q > 0)
                def _():
                    wait_scatter(r)

                compute(r)
                start_scatter(b, r)

                @pl.when(b + NR < NBLK1)
                def _():
                    start_gather(b + NR, r)

            return 0

        lax.fori_loop(0, NQ, quad, 0)
        for r in range(TAIL):
            wait_gather(r)
            wait_scatter(r)
            compute(r)
            start_scatter(NR * NQ + r, r)
        for r in range(NR):
            wait_scatter(r)

        plsc.subcore_barrier()
        pltpu.sync_copy(acc.at[pl.ds(s * RPT, RPT)],
                        out_hbm.at[c, pl.ds(s * RPT, RPT)])

        @pl.when(s == NS - 1)
        def _():
            pltpu.sync_copy(acc.at[pl.ds(NS * RPT, RTAIL)],
                            out_hbm.at[c, pl.ds(NS * RPT, RTAIL)])

    return k(h_mat, acat, src_r, dst_r)


def _edge_pass_sc_l2(h_mat, acat, src_r, dst_r):
    mesh = plsc.VectorSubcoreMesh(**_SC_MESH)

    @functools.partial(
        pl.kernel,
        out_type=jax.ShapeDtypeStruct((NC, N, 80), jnp.float32),
        mesh=mesh,
        compiler_params=_SC_PARAMS,
        scratch_types=[
            pltpu.VMEM((NBLK2, K2), jnp.int32),
            pltpu.VMEM((NBLK2, K2), jnp.int32),
            pltpu.VMEM((K2, 16), jnp.float32),
            pltpu.VMEM((K2, 16), jnp.float32),
            pltpu.VMEM((K2, 80), jnp.float32),
            pltpu.VMEM((K2, 80), jnp.float32),
            pltpu.VMEM((K2, 80), jnp.float32),
            pltpu.VMEM((K2, 80), jnp.float32),
            pltpu.VMEM_SHARED((N, 80), jnp.float32),
        ] + [pltpu.SemaphoreType.DMA] * 6,
    )
    def k(h_hbm, acat_hbm, srcr_hbm, dstr_hbm, out_hbm,
          srcv, dstv, bvA, bvB, rowsA, rowsB, stageA, stageB, acc,
          grA, gbA, grB, gbB, smA, smB):
        c = lax.axis_index("c")
        s = lax.axis_index("s")
        wid = s * NC + c
        lane = lax.iota(jnp.int32, 16)

        zv = jnp.zeros((16,), jnp.float32)

        def zbuf(i, _):
            for j in range(5):
                stageA[i, pl.ds(j * 16, 16)] = zv
                stageB[i, pl.ds(j * 16, 16)] = zv
            return 0

        lax.fori_loop(0, K2, zbuf, 0)
        for t in range(7):
            pltpu.sync_copy(stageA, acc.at[pl.ds(s * RPT + t * K2, K2)])
        pltpu.sync_copy(stageA.at[pl.ds(0, RPT - 7 * K2)],
                        acc.at[pl.ds(s * RPT + 7 * K2, RPT - 7 * K2)])

        @pl.when(s == NS - 1)
        def _():
            pltpu.sync_copy(stageA.at[pl.ds(0, RTAIL)],
                            acc.at[pl.ds(NS * RPT, RTAIL)])

        plsc.subcore_barrier()

        pltpu.sync_copy(srcr_hbm.at[wid], srcv)
        pltpu.sync_copy(dstr_hbm.at[wid], dstv)

        def start_gather(b, rows, bv, gr, gb):
            pltpu.async_copy(h_hbm.at[srcv.at[b]], rows, gr)
            pltpu.async_copy(acat_hbm.at[dstv.at[b]], bv, gb)

        def wait_gather(rows, bv, gr, gb):
            pltpu.make_async_copy(h_hbm.at[pl.ds(0, K2)], rows, gr).wait()
            pltpu.make_async_copy(acat_hbm.at[pl.ds(0, K2)], bv, gb).wait()

        def start_scatter(b, stage, sm):
            pltpu.async_copy(stage, acc.at[dstv.at[b]], sm, add=True)

        def wait_scatter(stage, sm):
            pltpu.make_async_copy(stage, acc.at[pl.ds(0, K2)], sm).wait()

        def compute(rows, bv, stage):
            def blk(i, _):
                eids = lane + 16 * i
                asr = plsc.load_gather(rows, [eids, lane * 0 + 64])
                ads = plsc.load_gather(bv, [eids, lane * 0 + 1])
                e = asr + ads
                e = jnp.where(e >= 0, e, e * 0.2)
                w = jnp.exp(e)
                plsc.store_scatter(stage, [eids, lane * 0 + 64], w)
                for d in range(16):
                    ii = i * 16 + d
                    w_s = w[d]
                    for j in range(4):
                        stage[ii, pl.ds(j * 16, 16)] = (
                            rows[ii, pl.ds(j * 16, 16)] * w_s)
                return 0

            lax.fori_loop(0, K2 // 16, blk, 0)

        start_gather(0, rowsA, bvA, grA, gbA)
        start_gather(1, rowsB, bvB, grB, gbB)

        NP = NBLK2 // 2

        def pair(p, _):
            a = 2 * p
            wait_gather(rowsA, bvA, grA, gbA)

            @pl.when(p > 0)
            def _():
                wait_scatter(stageA, smA)

            compute(rowsA, bvA, stageA)
            start_scatter(a, stageA, smA)
            start_gather(a + 2, rowsA, bvA, grA, gbA)

            wait_gather(rowsB, bvB, grB, gbB)

            @pl.when(p > 0)
            def _():
                wait_scatter(stageB, smB)

            compute(rowsB, bvB, stageB)
            start_scatter(a + 1, stageB, smB)

            @pl.when(p < NP - 1)
            def _():
                start_gather(a + 3, rowsB, bvB, grB, gbB)

            return 0

        lax.fori_loop(0, NP, pair, 0)

        wait_gather(rowsA, bvA, grA, gbA)
        wait_scatter(stageA, smA)
        compute(rowsA, bvA, stageA)
        start_scatter(NBLK2 - 1, stageA, smA)
        wait_scatter(stageA, smA)
        wait_scatter(stageB, smB)

        plsc.subcore_barrier()
        pltpu.sync_copy(acc.at[pl.ds(s * RPT, RPT)],
                        out_hbm.at[c, pl.ds(s * RPT, RPT)])

        @pl.when(s == NS - 1)
        def _():
            pltpu.sync_copy(acc.at[pl.ds(NS * RPT, RTAIL)],
                            out_hbm.at[c, pl.ds(NS * RPT, RTAIL)])

    accp = k(h_mat, acat, src_r, dst_r)
    return accp[0], accp[1]


def _edge_pass_jax(h, acat, src, dst, heads, ch, d_acc):
    a_src = acat[:, :heads]
    a_dst = acat[:, heads : 2 * heads]
    e = a_src[src] + a_dst[dst]
    e = jnp.where(e >= 0, e, 0.2 * e)
    w = jnp.exp(e)
    msg = (h[src].reshape(-1, heads, ch) * w[:, :, None]).reshape(-1, heads * ch)
    stage = jnp.concatenate(
        [msg, w, jnp.zeros((w.shape[0], d_acc - heads * ch - heads), jnp.float32)],
        axis=1,
    )
    acc = jax.ops.segment_sum(stage, dst, num_segments=N)
    return acc, jnp.zeros_like(acc)


def kernel(x, edge_index, W1, att_src1, att_dst1, b1, W2, att_src2, att_dst2, b2):
    src_r1 = edge_index[0].reshape(NW, NBLK1, K1)
    dst_r1 = edge_index[1].reshape(NW, NBLK1, K1)
    src_r2 = edge_index[0].reshape(NW, NBLK2, K2)
    dst_r2 = edge_index[1].reshape(NW, NBLK2, K2)
    h1, acat1 = _tc1(x, W1, att_src1, att_dst1)
    accp = _edge_pass_sc_l1(h1, acat1, src_r1, dst_r1)
    h2, acat2 = _tc2(accp[0], accp[1], b1, W2, att_src2, att_dst2)
    acc20, acc21 = _edge_pass_sc_l2(h2, acat2, src_r2, dst_r2)
    return _tc3(acc20, acc21, b2)


# speedup vs baseline: 1.2536x; 1.2536x over previous
import functools

import jax
import jax.numpy as jnp
from jax import lax
from jax.experimental import pallas as pl
from jax.experimental.pallas import tpu as pltpu
from jax.experimental.pallas import tpu_sc as plsc

N = 10000
E = 320000
IN_CH = 128
HID = 16
HEADS = 8
OUT_CH = 64

NB = 400


def _head_expand_mat(att):
    hh, cc = att.shape
    hc = hh * cc
    j = lax.broadcasted_iota(jnp.int32, (hc, hh), 0)
    k = lax.broadcasted_iota(jnp.int32, (hc, hh), 1)
    return jnp.where(j // cc == k, att.reshape(hc)[:, None], 0.0)


def _tc1_body(x_ref, w_ref, msrc_ref, mdst_ref, h_ref, acat_ref):
    xb = x_ref[...]
    h1 = xb @ w_ref[...]
    a_src = h1 @ msrc_ref[...]
    a_dst = h1 @ mdst_ref[...]
    h_ref[...] = jnp.concatenate([h1, a_src], axis=1)
    acat_ref[...] = jnp.concatenate([a_src, a_dst], axis=1)


def _tc1(x, W1, att_src1, att_dst1):
    msrc = _head_expand_mat(att_src1)
    mdst = _head_expand_mat(att_dst1)
    grid = (N // NB,)
    return pl.pallas_call(
        _tc1_body,
        grid=grid,
        in_specs=[
            pl.BlockSpec((NB, IN_CH), lambda i: (i, 0)),
            pl.BlockSpec((IN_CH, HEADS * HID), lambda i: (0, 0)),
            pl.BlockSpec((HEADS * HID, HEADS), lambda i: (0, 0)),
            pl.BlockSpec((HEADS * HID, HEADS), lambda i: (0, 0)),
        ],
        out_specs=[
            pl.BlockSpec((NB, HEADS * HID + 8), lambda i: (i, 0)),
            pl.BlockSpec((NB, 2 * HEADS), lambda i: (i, 0)),
        ],
        out_shape=[
            jax.ShapeDtypeStruct((N, HEADS * HID + 8), jnp.float32),
            jax.ShapeDtypeStruct((N, 2 * HEADS), jnp.float32),
        ],
    )(x, W1, msrc, mdst)


def _tc2_body(a0_ref, a1_ref, b1_ref, w2_ref, as2_ref, ad2_ref,
              h2_ref, acat2_ref):
    acc = a0_ref[...] + a1_ref[...]
    msg = acc[:, : HEADS * HID]
    wsum = acc[:, HEADS * HID : HEADS * HID + HEADS]
    j = lax.broadcasted_iota(jnp.int32, (HEADS, HEADS * HID), 1)
    hh = lax.broadcasted_iota(jnp.int32, (HEADS, HEADS * HID), 0)
    rep = jnp.where(j // HID == hh, 1.0, 0.0)
    denom = wsum @ rep
    x1 = msg / (denom + 1e-16) + b1_ref[...][None, :]
    xe = jnp.where(x1 > 0, x1, jnp.exp(x1) - 1.0)
    h2 = xe @ w2_ref[...]
    a2s = h2 @ as2_ref[...]
    a2d = h2 @ ad2_ref[...]
    pad = jnp.zeros((a2s.shape[0], 14), jnp.float32)
    h2_ref[...] = jnp.concatenate([h2, a2s, a2d, pad], axis=1)
    acat2_ref[...] = jnp.concatenate([a2s, a2d, pad], axis=1)


def _tc2(acc0, acc1, b1, W2, att_src2, att_dst2):
    grid = (N // NB,)
    d1 = HEADS * HID + HEADS
    return pl.pallas_call(
        _tc2_body,
        grid=grid,
        in_specs=[
            pl.BlockSpec((NB, d1), lambda i: (i, 0)),
            pl.BlockSpec((NB, d1), lambda i: (i, 0)),
            pl.BlockSpec((HEADS * HID,), lambda i: (0,)),
            pl.BlockSpec((HEADS * HID, OUT_CH), lambda i: (0, 0)),
            pl.BlockSpec((OUT_CH, 1), lambda i: (0, 0)),
            pl.BlockSpec((OUT_CH, 1), lambda i: (0, 0)),
        ],
        out_specs=[
            pl.BlockSpec((NB, OUT_CH + 16), lambda i: (i, 0)),
            pl.BlockSpec((NB, 16), lambda i: (i, 0)),
        ],
        out_shape=[
            jax.ShapeDtypeStruct((N, OUT_CH + 16), jnp.float32),
            jax.ShapeDtypeStruct((N, 16), jnp.float32),
        ],
    )(acc0, acc1, b1, W2,
      att_src2.reshape(OUT_CH, 1), att_dst2.reshape(OUT_CH, 1))


def _tc3_body(a0_ref, a1_ref, b2_ref, out_ref):
    acc = a0_ref[...] + a1_ref[...]
    out = acc[:, :OUT_CH] / (acc[:, OUT_CH : OUT_CH + 1] + 1e-16) + b2_ref[...][None, :]
    m = jnp.max(out, axis=1, keepdims=True)
    lse = jnp.log(jnp.sum(jnp.exp(out - m), axis=1, keepdims=True)) + m
    out_ref[...] = out - lse


def _tc3(acc0, acc1, b2):
    grid = (N // NB,)
    d2 = OUT_CH + 16
    return pl.pallas_call(
        _tc3_body,
        grid=grid,
        in_specs=[
            pl.BlockSpec((NB, d2), lambda i: (i, 0)),
            pl.BlockSpec((NB, d2), lambda i: (i, 0)),
            pl.BlockSpec((OUT_CH,), lambda i: (0,)),
        ],
        out_specs=pl.BlockSpec((NB, OUT_CH), lambda i: (i, 0)),
        out_shape=jax.ShapeDtypeStruct((N, OUT_CH), jnp.float32),
    )(acc0, acc1, b2)


NC = 2
NS = 16
NW = NC * NS
K1 = 40
NBLK1 = E // (NW * K1)
K2 = 80
NBLK2 = E // (NW * K2)
RPT = 624
RTAIL = N - NS * RPT

_SC_MESH = dict(core_axis_name="c", subcore_axis_name="s")
_SC_PARAMS = pltpu.CompilerParams(use_tc_tiling_on_sc=False,
                                  needs_layout_passes=False)


def _edge_pass_sc_l1(h_mat, acat, src_r, dst_r):
    mesh = plsc.VectorSubcoreMesh(**_SC_MESH)
    NR = 4

    @functools.partial(
        pl.kernel,
        out_type=jax.ShapeDtypeStruct((NC, N, 136), jnp.float32),
        mesh=mesh,
        compiler_params=_SC_PARAMS,
        scratch_types=[
            pltpu.VMEM((NBLK1, K1), jnp.int32),
            pltpu.VMEM((NBLK1, K1), jnp.int32),
        ] + [pltpu.VMEM((K1, 136), jnp.float32)] * NR
          + [pltpu.VMEM((K1, 16), jnp.float32)] * NR
          + [pltpu.VMEM_SHARED((N, 136), jnp.float32)]
          + [pltpu.SemaphoreType.DMA] * (3 * NR),
    )
    def k(h_hbm, acat_hbm, srcr_hbm, dstr_hbm, out_hbm, srcv, dstv, *rest):
        rows = rest[0:NR]
        bvs = rest[NR:2 * NR]
        acc = rest[2 * NR]
        grs = rest[2 * NR + 1: 2 * NR + 1 + NR]
        gbs = rest[2 * NR + 1 + NR: 2 * NR + 1 + 2 * NR]
        sms = rest[2 * NR + 1 + 2 * NR: 2 * NR + 1 + 3 * NR]
        c = lax.axis_index("c")
        s = lax.axis_index("s")
        wid = s * NC + c
        lane = lax.iota(jnp.int32, 16)
        lane8 = lane // 8
        head8 = lane - 8 * lane8

        zv = jnp.zeros((16,), jnp.float32)

        def zbuf(i, _):
            for j in range(8):
                rows[0][i, pl.ds(j * 16, 16)] = zv
            rows[0][i, pl.ds(120, 16)] = zv
            return 0

        lax.fori_loop(0, K1, zbuf, 0)
        for t in range(15):
            pltpu.sync_copy(rows[0], acc.at[pl.ds(s * RPT + t * K1, K1)])
        pltpu.sync_copy(rows[0].at[pl.ds(0, RPT - 15 * K1)],
                        acc.at[pl.ds(s * RPT + 15 * K1, RPT - 15 * K1)])

        @pl.when(s == NS - 1)
        def _():
            pltpu.sync_copy(rows[0].at[pl.ds(0, RTAIL)],
                            acc.at[pl.ds(NS * RPT, RTAIL)])

        plsc.subcore_barrier()

        pltpu.sync_copy(srcr_hbm.at[wid], srcv)
        pltpu.sync_copy(dstr_hbm.at[wid], dstv)

        def start_gather(b, r):
            pltpu.async_copy(h_hbm.at[srcv.at[b]], rows[r], grs[r])
            pltpu.async_copy(acat_hbm.at[dstv.at[b]], bvs[r], gbs[r])

        def wait_gather(r):
            pltpu.make_async_copy(h_hbm.at[pl.ds(0, K1)], rows[r], grs[r]).wait()
            pltpu.make_async_copy(acat_hbm.at[pl.ds(0, K1)], bvs[r], gbs[r]).wait()

        def start_scatter(b, r):
            pltpu.async_copy(rows[r], acc.at[dstv.at[b]], sms[r], add=True)

        def wait_scatter(r):
            pltpu.make_async_copy(rows[r], acc.at[pl.ds(0, K1)], sms[r]).wait()

        def compute(r):
            rw = rows[r]
            bv = bvs[r]

            def blk(i2, _):
                eids = lane8 + 2 * i2
                asr = plsc.load_gather(rw, [eids, head8 + 128])
                ads = plsc.load_gather(bv, [eids, head8 + 8])
                e = asr + ads
                e = jnp.where(e >= 0, e, e * 0.2)
                w = jnp.exp(e)
                plsc.store_scatter(rw, [eids, head8 + 128], w)
                for d in range(2):
                    i = 2 * i2 + d
                    for j in range(8):
                        w_s = w[d * 8 + j]
                        rw[i, pl.ds(j * 16, 16)] = rw[i, pl.ds(j * 16, 16)] * w_s
                return 0

            lax.fori_loop(0, K1 // 2, blk, 0)

        for r in range(NR):
            start_gather(r, r)

        NQ = NBLK1 // NR
        TAIL = NBLK1 - NR * NQ

        def quad(q, _):
            for r in range(NR):
                b = NR * q + r
                wait_gather(r)

                @pl.when(q > 0)
                def _():
                    wait_scatter(r)

                compute(r)
                start_scatter(b, r)

                @pl.when(b + NR < NBLK1)
                def _():
                    start_gather(b + NR, r)

            return 0

        lax.fori_loop(0, NQ, quad, 0)
        for r in range(TAIL):
            wait_gather(r)
            wait_scatter(r)
            compute(r)
            start_scatter(NR * NQ + r, r)
        for r in range(NR):
            wait_scatter(r)

        plsc.subcore_barrier()
        pltpu.sync_copy(acc.at[pl.ds(s * RPT, RPT)],
                        out_hbm.at[c, pl.ds(s * RPT, RPT)])

        @pl.when(s == NS - 1)
        def _():
            pltpu.sync_copy(acc.at[pl.ds(NS * RPT, RTAIL)],
                            out_hbm.at[c, pl.ds(NS * RPT, RTAIL)])

    return k(h_mat, acat, src_r, dst_r)


def _edge_pass_sc_l2(h_mat, acat, src_r, dst_r):
    mesh = plsc.VectorSubcoreMesh(**_SC_MESH)

    @functools.partial(
        pl.kernel,
        out_type=jax.ShapeDtypeStruct((NC, N, 80), jnp.float32),
        mesh=mesh,
        compiler_params=_SC_PARAMS,
        scratch_types=[
            pltpu.VMEM((NBLK2, K2), jnp.int32),
            pltpu.VMEM((NBLK2, K2), jnp.int32),
            pltpu.VMEM((K2, 16), jnp.float32),
            pltpu.VMEM((K2, 16), jnp.float32),
            pltpu.VMEM((K2, 80), jnp.float32),
            pltpu.VMEM((K2, 80), jnp.float32),
            pltpu.VMEM((K2,), jnp.float32),
            pltpu.VMEM((K2, 80), jnp.float32),
            pltpu.VMEM((K2, 80), jnp.float32),
            pltpu.VMEM_SHARED((N, 80), jnp.float32),
        ] + [pltpu.SemaphoreType.DMA] * 6,
    )
    def k(h_hbm, acat_hbm, srcr_hbm, dstr_hbm, out_hbm,
          srcv, dstv, bvA, bvB, rowsA, rowsB, wv, stageA, stageB, acc,
          grA, gbA, grB, gbB, smA, smB):
        c = lax.axis_index("c")
        s = lax.axis_index("s")
        wid = s * NC + c
        lane = lax.iota(jnp.int32, 16)

        zv = jnp.zeros((16,), jnp.float32)

        def zbuf(i, _):
            for j in range(5):
                stageA[i, pl.ds(j * 16, 16)] = zv
                stageB[i, pl.ds(j * 16, 16)] = zv
            return 0

        lax.fori_loop(0, K2, zbuf, 0)
        for t in range(7):
            pltpu.sync_copy(stageA, acc.at[pl.ds(s * RPT + t * K2, K2)])
        pltpu.sync_copy(stageA.at[pl.ds(0, RPT - 7 * K2)],
                        acc.at[pl.ds(s * RPT + 7 * K2, RPT - 7 * K2)])

        @pl.when(s == NS - 1)
        def _():
            pltpu.sync_copy(stageA.at[pl.ds(0, RTAIL)],
                            acc.at[pl.ds(NS * RPT, RTAIL)])

        plsc.subcore_barrier()

        pltpu.sync_copy(srcr_hbm.at[wid], srcv)
        pltpu.sync_copy(dstr_hbm.at[wid], dstv)

        def start_gather(b, rows, bv, gr, gb):
            pltpu.async_copy(h_hbm.at[srcv.at[b]], rows, gr)
            pltpu.async_copy(acat_hbm.at[dstv.at[b]], bv, gb)

        def wait_gather(rows, bv, gr, gb):
            pltpu.make_async_copy(h_hbm.at[pl.ds(0, K2)], rows, gr).wait()
            pltpu.make_async_copy(acat_hbm.at[pl.ds(0, K2)], bv, gb).wait()

        def start_scatter(b, stage, sm):
            pltpu.async_copy(stage, acc.at[dstv.at[b]], sm, add=True)

        def wait_scatter(stage, sm):
            pltpu.make_async_copy(stage, acc.at[pl.ds(0, K2)], sm).wait()

        def compute(rows, bv, stage):
            def wblk(i, _):
                eids = lane + 16 * i
                asr = plsc.load_gather(rows, [eids, lane * 0 + 64])
                ads = plsc.load_gather(bv, [eids, lane * 0 + 1])
                e = asr + ads
                e = jnp.where(e >= 0, e, e * 0.2)
                w = jnp.exp(e)
                wv[pl.ds(i * 16, 16)] = w
                plsc.store_scatter(stage, [eids, lane * 0 + 64], w)
                return 0

            lax.fori_loop(0, K2 // 16, wblk, 0)

            def mblk(t, _):
                w16 = wv[pl.ds(t * 16, 16)]
                for d in range(16):
                    i = t * 16 + d
                    w_s = w16[d]
                    for j in range(4):
                        stage[i, pl.ds(j * 16, 16)] = (
                            rows[i, pl.ds(j * 16, 16)] * w_s)
                return 0

            lax.fori_loop(0, K2 // 16, mblk, 0)

        start_gather(0, rowsA, bvA, grA, gbA)
        start_gather(1, rowsB, bvB, grB, gbB)

        NP = NBLK2 // 2

        def pair(p, _):
            a = 2 * p
            wait_gather(rowsA, bvA, grA, gbA)

            @pl.when(p > 0)
            def _():
                wait_scatter(stageA, smA)

            compute(rowsA, bvA, stageA)
            start_scatter(a, stageA, smA)
            start_gather(a + 2, rowsA, bvA, grA, gbA)

            wait_gather(rowsB, bvB, grB, gbB)

            @pl.when(p > 0)
            def _():
                wait_scatter(stageB, smB)

            compute(rowsB, bvB, stageB)
            start_scatter(a + 1, stageB, smB)

            @pl.when(p < NP - 1)
            def _():
                start_gather(a + 3, rowsB, bvB, grB, gbB)

            return 0

        lax.fori_loop(0, NP, pair, 0)

        wait_gather(rowsA, bvA, grA, gbA)
        wait_scatter(stageA, smA)
        compute(rowsA, bvA, stageA)
        start_scatter(NBLK2 - 1, stageA, smA)
        wait_scatter(stageA, smA)
        wait_scatter(stageB, smB)

        plsc.subcore_barrier()
        pltpu.sync_copy(acc.at[pl.ds(s * RPT, RPT)],
                        out_hbm.at[c, pl.ds(s * RPT, RPT)])

        @pl.when(s == NS - 1)
        def _():
            pltpu.sync_copy(acc.at[pl.ds(NS * RPT, RTAIL)],
                            out_hbm.at[c, pl.ds(NS * RPT, RTAIL)])

    accp = k(h_mat, acat, src_r, dst_r)
    return accp[0], accp[1]


def _edge_pass_jax(h, acat, src, dst, heads, ch, d_acc):
    a_src = acat[:, :heads]
    a_dst = acat[:, heads : 2 * heads]
    e = a_src[src] + a_dst[dst]
    e = jnp.where(e >= 0, e, 0.2 * e)
    w = jnp.exp(e)
    msg = (h[src].reshape(-1, heads, ch) * w[:, :, None]).reshape(-1, heads * ch)
    stage = jnp.concatenate(
        [msg, w, jnp.zeros((w.shape[0], d_acc - heads * ch - heads), jnp.float32)],
        axis=1,
    )
    acc = jax.ops.segment_sum(stage, dst, num_segments=N)
    return acc, jnp.zeros_like(acc)


def kernel(x, edge_index, W1, att_src1, att_dst1, b1, W2, att_src2, att_dst2, b2):
    src_r1 = edge_index[0].reshape(NW, NBLK1, K1)
    dst_r1 = edge_index[1].reshape(NW, NBLK1, K1)
    src_r2 = edge_index[0].reshape(NW, NBLK2, K2)
    dst_r2 = edge_index[1].reshape(NW, NBLK2, K2)
    h1, acat1 = _tc1(x, W1, att_src1, att_dst1)
    accp = _edge_pass_sc_l1(h1, acat1, src_r1, dst_r1)
    h2, acat2 = _tc2(accp[0], accp[1], b1, W2, att_src2, att_dst2)
    acc20, acc21 = _edge_pass_sc_l2(h2, acat2, src_r2, dst_r2)
    return _tc3(acc20, acc21, b2)


# speedup vs baseline: 1.2546x; 1.0009x over previous
import functools

import jax
import jax.numpy as jnp
from jax import lax
from jax.experimental import pallas as pl
from jax.experimental.pallas import tpu as pltpu
from jax.experimental.pallas import tpu_sc as plsc

N = 10000
E = 320000
IN_CH = 128
HID = 16
HEADS = 8
OUT_CH = 64

NB = 400


def _head_expand_mat(att):
    hh, cc = att.shape
    hc = hh * cc
    j = lax.broadcasted_iota(jnp.int32, (hc, hh), 0)
    k = lax.broadcasted_iota(jnp.int32, (hc, hh), 1)
    return jnp.where(j // cc == k, att.reshape(hc)[:, None], 0.0)


def _tc1_body(x_ref, w_ref, msrc_ref, mdst_ref, h_ref, acat_ref):
    xb = x_ref[...]
    h1 = xb @ w_ref[...]
    a_src = h1 @ msrc_ref[...]
    a_dst = h1 @ mdst_ref[...]
    h_ref[...] = jnp.concatenate([h1, a_src], axis=1)
    acat_ref[...] = jnp.concatenate([a_src, a_dst], axis=1)


def _tc1(x, W1, att_src1, att_dst1):
    msrc = _head_expand_mat(att_src1)
    mdst = _head_expand_mat(att_dst1)
    grid = (N // NB,)
    return pl.pallas_call(
        _tc1_body,
        grid=grid,
        in_specs=[
            pl.BlockSpec((NB, IN_CH), lambda i: (i, 0)),
            pl.BlockSpec((IN_CH, HEADS * HID), lambda i: (0, 0)),
            pl.BlockSpec((HEADS * HID, HEADS), lambda i: (0, 0)),
            pl.BlockSpec((HEADS * HID, HEADS), lambda i: (0, 0)),
        ],
        out_specs=[
            pl.BlockSpec((NB, HEADS * HID + 8), lambda i: (i, 0)),
            pl.BlockSpec((NB, 2 * HEADS), lambda i: (i, 0)),
        ],
        out_shape=[
            jax.ShapeDtypeStruct((N, HEADS * HID + 8), jnp.float32),
            jax.ShapeDtypeStruct((N, 2 * HEADS), jnp.float32),
        ],
    )(x, W1, msrc, mdst)


def _tc2_body(a0_ref, a1_ref, b1_ref, w2_ref, as2_ref, ad2_ref,
              h2_ref, acat2_ref):
    acc = a0_ref[...] + a1_ref[...]
    msg = acc[:, : HEADS * HID]
    wsum = acc[:, HEADS * HID : HEADS * HID + HEADS]
    j = lax.broadcasted_iota(jnp.int32, (HEADS, HEADS * HID), 1)
    hh = lax.broadcasted_iota(jnp.int32, (HEADS, HEADS * HID), 0)
    rep = jnp.where(j // HID == hh, 1.0, 0.0)
    denom = wsum @ rep
    x1 = msg / (denom + 1e-16) + b1_ref[...][None, :]
    xe = jnp.where(x1 > 0, x1, jnp.exp(x1) - 1.0)
    h2 = xe @ w2_ref[...]
    a2s = h2 @ as2_ref[...]
    a2d = h2 @ ad2_ref[...]
    pad = jnp.zeros((a2s.shape[0], 14), jnp.float32)
    h2_ref[...] = jnp.concatenate([h2, a2s, a2d, pad], axis=1)
    acat2_ref[...] = jnp.concatenate([a2s, a2d, pad], axis=1)


def _tc2(acc0, acc1, b1, W2, att_src2, att_dst2):
    grid = (N // NB,)
    d1 = HEADS * HID + HEADS
    return pl.pallas_call(
        _tc2_body,
        grid=grid,
        in_specs=[
            pl.BlockSpec((NB, d1), lambda i: (i, 0)),
            pl.BlockSpec((NB, d1), lambda i: (i, 0)),
            pl.BlockSpec((HEADS * HID,), lambda i: (0,)),
            pl.BlockSpec((HEADS * HID, OUT_CH), lambda i: (0, 0)),
            pl.BlockSpec((OUT_CH, 1), lambda i: (0, 0)),
            pl.BlockSpec((OUT_CH, 1), lambda i: (0, 0)),
        ],
        out_specs=[
            pl.BlockSpec((NB, OUT_CH + 16), lambda i: (i, 0)),
            pl.BlockSpec((NB, 16), lambda i: (i, 0)),
        ],
        out_shape=[
            jax.ShapeDtypeStruct((N, OUT_CH + 16), jnp.float32),
            jax.ShapeDtypeStruct((N, 16), jnp.float32),
        ],
    )(acc0, acc1, b1, W2,
      att_src2.reshape(OUT_CH, 1), att_dst2.reshape(OUT_CH, 1))


def _tc3_body(a0_ref, a1_ref, b2_ref, out_ref):
    acc = a0_ref[...] + a1_ref[...]
    out = acc[:, :OUT_CH] / (acc[:, OUT_CH : OUT_CH + 1] + 1e-16) + b2_ref[...][None, :]
    m = jnp.max(out, axis=1, keepdims=True)
    lse = jnp.log(jnp.sum(jnp.exp(out - m), axis=1, keepdims=True)) + m
    out_ref[...] = out - lse


def _tc3(acc0, acc1, b2):
    grid = (N // NB,)
    d2 = OUT_CH + 16
    return pl.pallas_call(
        _tc3_body,
        grid=grid,
        in_specs=[
            pl.BlockSpec((NB, d2), lambda i: (i, 0)),
            pl.BlockSpec((NB, d2), lambda i: (i, 0)),
            pl.BlockSpec((OUT_CH,), lambda i: (0,)),
        ],
        out_specs=pl.BlockSpec((NB, OUT_CH), lambda i: (i, 0)),
        out_shape=jax.ShapeDtypeStruct((N, OUT_CH), jnp.float32),
    )(acc0, acc1, b2)


NC = 2
NS = 16
NW = NC * NS
K1 = 40
NBLK1 = E // (NW * K1)
K2 = 80
NBLK2 = E // (NW * K2)
RPT = 624
RTAIL = N - NS * RPT

_SC_MESH = dict(core_axis_name="c", subcore_axis_name="s")
_SC_PARAMS = pltpu.CompilerParams(use_tc_tiling_on_sc=False,
                                  needs_layout_passes=False)


def _edge_pass_sc_l1(h_mat, acat, src_r, dst_r):
    mesh = plsc.VectorSubcoreMesh(**_SC_MESH)
    NR = 4

    @functools.partial(
        pl.kernel,
        out_type=jax.ShapeDtypeStruct((NC, N, 136), jnp.float32),
        mesh=mesh,
        compiler_params=_SC_PARAMS,
        scratch_types=[
            pltpu.VMEM((NBLK1, K1), jnp.int32),
            pltpu.VMEM((NBLK1, K1), jnp.int32),
        ] + [pltpu.VMEM((K1, 136), jnp.float32)] * NR
          + [pltpu.VMEM((K1, 16), jnp.float32)] * NR
          + [pltpu.VMEM_SHARED((N, 136), jnp.float32)]
          + [pltpu.SemaphoreType.DMA] * (3 * NR),
    )
    def k(h_hbm, acat_hbm, srcr_hbm, dstr_hbm, out_hbm, srcv, dstv, *rest):
        rows = rest[0:NR]
        bvs = rest[NR:2 * NR]
        acc = rest[2 * NR]
        grs = rest[2 * NR + 1: 2 * NR + 1 + NR]
        gbs = rest[2 * NR + 1 + NR: 2 * NR + 1 + 2 * NR]
        sms = rest[2 * NR + 1 + 2 * NR: 2 * NR + 1 + 3 * NR]
        c = lax.axis_index("c")
        s = lax.axis_index("s")
        wid = s * NC + c
        lane = lax.iota(jnp.int32, 16)
        lane8 = lane // 8
        head8 = lane - 8 * lane8

        zv = jnp.zeros((16,), jnp.float32)

        def zbuf(i, _):
            for j in range(8):
                rows[0][i, pl.ds(j * 16, 16)] = zv
            rows[0][i, pl.ds(120, 16)] = zv
            return 0

        lax.fori_loop(0, K1, zbuf, 0)
        for t in range(15):
            pltpu.sync_copy(rows[0], acc.at[pl.ds(s * RPT + t * K1, K1)])
        pltpu.sync_copy(rows[0].at[pl.ds(0, RPT - 15 * K1)],
                        acc.at[pl.ds(s * RPT + 15 * K1, RPT - 15 * K1)])

        @pl.when(s == NS - 1)
        def _():
            pltpu.sync_copy(rows[0].at[pl.ds(0, RTAIL)],
                            acc.at[pl.ds(NS * RPT, RTAIL)])

        plsc.subcore_barrier()

        pltpu.sync_copy(srcr_hbm.at[wid], srcv)
        pltpu.sync_copy(dstr_hbm.at[wid], dstv)

        def start_gather(b, r):
            pltpu.async_copy(h_hbm.at[srcv.at[b]], rows[r], grs[r])
            pltpu.async_copy(acat_hbm.at[dstv.at[b]], bvs[r], gbs[r])

        def wait_gather(r):
            pltpu.make_async_copy(h_hbm.at[pl.ds(0, K1)], rows[r], grs[r]).wait()
            pltpu.make_async_copy(acat_hbm.at[pl.ds(0, K1)], bvs[r], gbs[r]).wait()

        def start_scatter(b, r):
            pltpu.async_copy(rows[r], acc.at[dstv.at[b]], sms[r], add=True)

        def wait_scatter(r):
            pltpu.make_async_copy(rows[r], acc.at[pl.ds(0, K1)], sms[r]).wait()

        def compute(r):
            rw = rows[r]
            bv = bvs[r]

            def blk(i2, _):
                eids = lane8 + 2 * i2
                asr = plsc.load_gather(rw, [eids, head8 + 128])
                ads = plsc.load_gather(bv, [eids, head8 + 8])
                e = asr + ads
                e = jnp.where(e >= 0, e, e * 0.2)
                w = jnp.exp(e)
                plsc.store_scatter(rw, [eids, head8 + 128], w)
                for d in range(2):
                    i = 2 * i2 + d
                    for j in range(8):
                        w_s = w[d * 8 + j]
                        rw[i, pl.ds(j * 16, 16)] = rw[i, pl.ds(j * 16, 16)] * w_s
                return 0

            lax.fori_loop(0, K1 // 2, blk, 0)

        for r in range(NR):
            start_gather(r, r)

        NQ = NBLK1 // NR
        TAIL = NBLK1 - NR * NQ

        def quad(q, _):
            for r in range(NR):
                b = NR * q + r
                wait_gather(r)

                @pl.when(q > 0)
                def _():
                    wait_scatter(r)

                compute(r)
                start_scatter(b, r)

                @pl.when(b + NR < NBLK1)
                def _():
                    start_gather(b + NR, r)

            return 0

        lax.fori_loop(0, NQ, quad, 0)
        for r in range(TAIL):
            wait_gather(r)
            wait_scatter(r)
            compute(r)
            start_scatter(NR * NQ + r, r)
        for r in range(NR):
            wait_scatter(r)

        plsc.subcore_barrier()
        pltpu.sync_copy(acc.at[pl.ds(s * RPT, RPT)],
                        out_hbm.at[c, pl.ds(s * RPT, RPT)])

        @pl.when(s == NS - 1)
        def _():
            pltpu.sync_copy(acc.at[pl.ds(NS * RPT, RTAIL)],
                            out_hbm.at[c, pl.ds(NS * RPT, RTAIL)])

    return k(h_mat, acat, src_r, dst_r)


def _edge_pass_sc_l2(h_mat, acat, src_r, dst_r):
    mesh = plsc.VectorSubcoreMesh(**_SC_MESH)

    @functools.partial(
        pl.kernel,
        out_type=jax.ShapeDtypeStruct((NC, N, 80), jnp.float32),
        mesh=mesh,
        compiler_params=_SC_PARAMS,
        scratch_types=[
            pltpu.VMEM((NBLK2, K2), jnp.int32),
            pltpu.VMEM((NBLK2, K2), jnp.int32),
            pltpu.VMEM((K2, 16), jnp.float32),
            pltpu.VMEM((K2, 16), jnp.float32),
            pltpu.VMEM((K2, 80), jnp.float32),
            pltpu.VMEM((K2, 80), jnp.float32),
            pltpu.VMEM((K2,), jnp.float32),
            pltpu.VMEM((K2, 80), jnp.float32),
            pltpu.VMEM((K2, 80), jnp.float32),
            pltpu.VMEM_SHARED((N, 80), jnp.float32),
        ] + [pltpu.SemaphoreType.DMA] * 6,
    )
    def k(h_hbm, acat_hbm, srcr_hbm, dstr_hbm, out_hbm,
          srcv, dstv, bvA, bvB, rowsA, rowsB, wv, stageA, stageB, acc,
          grA, gbA, grB, gbB, smA, smB):
        c = lax.axis_index("c")
        s = lax.axis_index("s")
        wid = s * NC + c
        lane = lax.iota(jnp.int32, 16)

        zv = jnp.zeros((16,), jnp.float32)

        def zbuf(i, _):
            for j in range(5):
                stageA[i, pl.ds(j * 16, 16)] = zv
                stageB[i, pl.ds(j * 16, 16)] = zv
            return 0

        lax.fori_loop(0, K2, zbuf, 0)
        for t in range(7):
            pltpu.sync_copy(stageA, acc.at[pl.ds(s * RPT + t * K2, K2)])
        pltpu.sync_copy(stageA.at[pl.ds(0, RPT - 7 * K2)],
                        acc.at[pl.ds(s * RPT + 7 * K2, RPT - 7 * K2)])

        @pl.when(s == NS - 1)
        def _():
            pltpu.sync_copy(stageA.at[pl.ds(0, RTAIL)],
                            acc.at[pl.ds(NS * RPT, RTAIL)])

        plsc.subcore_barrier()

        pltpu.sync_copy(srcr_hbm.at[wid], srcv)
        pltpu.sync_copy(dstr_hbm.at[wid], dstv)

        def start_gather(b, rows, bv, gr, gb):
            pltpu.async_copy(h_hbm.at[srcv.at[b]], rows, gr)
            pltpu.async_copy(acat_hbm.at[dstv.at[b]], bv, gb)

        def wait_gather(rows, bv, gr, gb):
            pltpu.make_async_copy(h_hbm.at[pl.ds(0, K2)], rows, gr).wait()
            pltpu.make_async_copy(acat_hbm.at[pl.ds(0, K2)], bv, gb).wait()

        def start_scatter(b, stage, sm):
            pltpu.async_copy(stage, acc.at[dstv.at[b]], sm, add=True)

        def wait_scatter(stage, sm):
            pltpu.make_async_copy(stage, acc.at[pl.ds(0, K2)], sm).wait()

        def compute(rows, bv, stage):
            def wblk(i, _):
                eids = lane + 16 * i
                asr = plsc.load_gather(rows, [eids, lane * 0 + 64])
                ads = plsc.load_gather(bv, [eids, lane * 0 + 1])
                e = asr + ads
                e = jnp.where(e >= 0, e, e * 0.2)
                w = jnp.exp(e)
                wv[pl.ds(i * 16, 16)] = w
                plsc.store_scatter(stage, [eids, lane * 0 + 64], w)
                return 0

            lax.fori_loop(0, K2 // 16, wblk, 0)

            def mblk(t, _):
                w16 = wv[pl.ds(t * 16, 16)]
                for d in range(16):
                    i = t * 16 + d
                    w_s = w16[d]
                    for j in range(4):
                        stage[i, pl.ds(j * 16, 16)] = (
                            rows[i, pl.ds(j * 16, 16)] * w_s)
                return 0

            lax.fori_loop(0, K2 // 16, mblk, 0)

        start_gather(0, rowsA, bvA, grA, gbA)
        start_gather(1, rowsB, bvB, grB, gbB)

        NP = NBLK2 // 2

        def pair(p, _):
            a = 2 * p
            wait_gather(rowsA, bvA, grA, gbA)

            @pl.when(p > 0)
            def _():
                wait_scatter(stageA, smA)

            compute(rowsA, bvA, stageA)
            start_scatter(a, stageA, smA)
            start_gather(a + 2, rowsA, bvA, grA, gbA)

            wait_gather(rowsB, bvB, grB, gbB)

            @pl.when(p > 0)
            def _():
                wait_scatter(stageB, smB)

            compute(rowsB, bvB, stageB)
            start_scatter(a + 1, stageB, smB)

            @pl.when(p < NP - 1)
            def _():
                start_gather(a + 3, rowsB, bvB, grB, gbB)

            return 0

        lax.fori_loop(0, NP, pair, 0)

        wait_gather(rowsA, bvA, grA, gbA)
        wait_scatter(stageA, smA)
        compute(rowsA, bvA, stageA)
        start_scatter(NBLK2 - 1, stageA, smA)
        wait_scatter(stageA, smA)
        wait_scatter(stageB, smB)

        plsc.subcore_barrier()
        pltpu.sync_copy(acc.at[pl.ds(s * RPT, RPT)],
                        out_hbm.at[c, pl.ds(s * RPT, RPT)])

        @pl.when(s == NS - 1)
        def _():
            pltpu.sync_copy(acc.at[pl.ds(NS * RPT, RTAIL)],
                            out_hbm.at[c, pl.ds(NS * RPT, RTAIL)])

    accp = k(h_mat, acat, src_r, dst_r)
    return accp[0], accp[1]


def kernel(x, edge_index, W1, att_src1, att_dst1, b1, W2, att_src2, att_dst2, b2):
    src_r1 = edge_index[0].reshape(NW, NBLK1, K1)
    dst_r1 = edge_index[1].reshape(NW, NBLK1, K1)
    src_r2 = edge_index[0].reshape(NW, NBLK2, K2)
    dst_r2 = edge_index[1].reshape(NW, NBLK2, K2)
    h1, acat1 = _tc1(x, W1, att_src1, att_dst1)
    accp = _edge_pass_sc_l1(h1, acat1, src_r1, dst_r1)
    h2, acat2 = _tc2(accp[0], accp[1], b1, W2, att_src2, att_dst2)
    acc20, acc21 = _edge_pass_sc_l2(h2, acat2, src_r2, dst_r2)
    return _tc3(acc20, acc21, b2)
